# baseline (speedup 1.0000x reference)
import functools

import jax
import jax.numpy as jnp
from jax import lax
from jax.experimental import pallas as pl
from jax.experimental.pallas import tpu as pltpu
from jax.experimental.pallas import tpu_sc as plsc

B = 16384
ED = 64
NT = 5
NC, NS = 2, 16
NW = NC * NS
ROWS_W = B // NW
CHUNK = 128
NCH = ROWS_W // CHUNK

BBLK = 2048
NB = B // BBLK


def _sc_gather_kernel(idx_hbm, t0, t1, t2, t3, t4,
                      o0, o1, o2, o3, o4,
                      idx_v, rows_a, rows_b, sem):
    wid = lax.axis_index("s") * NC + lax.axis_index("c")
    base = wid * ROWS_W
    pltpu.sync_copy(idx_hbm.at[wid], idx_v)
    tbls = [t0, t1, t2, t3, t4]
    outs = [o0, o1, o2, o3, o4]
    bufs = [rows_a, rows_b]
    for t in range(NT):
        buf = bufs[t % 2]
        cps = [
            pltpu.async_copy(
                tbls[t].at[idx_v.at[t, j]],
                buf.at[pl.ds(j * CHUNK, CHUNK)],
                sem,
            )
            for j in range(NCH)
        ]
        for cp in cps:
            cp.wait()
        pltpu.sync_copy(buf, outs[t].at[pl.ds(base, ROWS_W)])


def _sc_gather(idx_stack, tables):
    mesh = plsc.VectorSubcoreMesh(core_axis_name="c", subcore_axis_name="s")
    out_type = tuple(
        jax.ShapeDtypeStruct((B, ED), jnp.float32) for _ in range(NT)
    )
    k = pl.kernel(
        _sc_gather_kernel,
        out_type=out_type,
        mesh=mesh,
        scratch_types=[
            pltpu.VMEM((NT, NCH, CHUNK), jnp.int32),
            pltpu.VMEM((ROWS_W, ED), jnp.float32),
            pltpu.VMEM((ROWS_W, ED), jnp.float32),
            pltpu.SemaphoreType.DMA,
        ],
        compiler_params=pltpu.CompilerParams(use_tc_tiling_on_sc=False),
    )
    return k(idx_stack, *tables)


def _mlp_body(g0, g1, g2, g3, g4, w1, b1, w2, b2, w3, b3, out):
    x = jnp.concatenate(
        [g0[...], g1[...], g2[...], g3[...], g4[...]], axis=1
    )
    h = jnp.dot(x, w1[...], preferred_element_type=jnp.float32,
                precision=lax.Precision.HIGHEST)
    h = jnp.maximum(h + b1[...], 0.0)
    h2 = jnp.dot(h, w2[...], preferred_element_type=jnp.float32,
                 precision=lax.Precision.HIGHEST)
    h2 = jnp.maximum(h2 + b2[...], 0.0)
    o = jnp.sum(h2 * w3[...], axis=1) + b3[0, 0]
    out[0, 0, :] = o


def _tc_mlp(gs, W1, b1, W2, b2, W3, b3):
    full = lambda shape: pl.BlockSpec(shape, lambda i: (0, 0))
    out = pl.pallas_call(
        _mlp_body,
        grid=(NB,),
        in_specs=[pl.BlockSpec((BBLK, ED), lambda i: (i, 0))] * NT + [
            full((NT * ED, ED)),
            full((1, ED)),
            full((ED, 32)),
            full((1, 32)),
            full((1, 32)),
            full((1, 1)),
        ],
        out_specs=pl.BlockSpec((1, 1, BBLK), lambda i: (i, 0, 0)),
        out_shape=jax.ShapeDtypeStruct((NB, 1, BBLK), jnp.float32),
    )(*gs, W1, b1.reshape(1, ED), W2, b2.reshape(1, 32),
      W3.reshape(1, 32), b3.reshape(1, 1))
    return out.reshape(-1)


def kernel(user, item, genre, country, tags,
           user_table, item_table, genre_table, country_table, tags_table,
           W1, b1, W2, b2, W3, b3):
    idx = jnp.stack([user, item, genre, country, tags]).astype(jnp.int32)
    idx = idx.reshape(NT, NW, NCH, CHUNK).transpose(1, 0, 2, 3)
    gs = _sc_gather(
        idx, (user_table, item_table, genre_table, country_table, tags_table)
    )
    return _tc_mlp(gs, W1, b1, W2, b2, W3, b3)


# speedup vs baseline: 1.5199x; 1.5199x over previous
import functools

import jax
import jax.numpy as jnp
from jax import lax
from jax.experimental import pallas as pl
from jax.experimental.pallas import tpu as pltpu
from jax.experimental.pallas import tpu_sc as plsc

B = 16384
ED = 64
NT = 5
NC, NS = 2, 16
NW = NC * NS
ROWS_W = B // NW
BUFR = 256

BBLK = 2048
NB = B // BBLK


def _sc_gather_kernel(idx_hbm, t0, t1, t2, t3, t4,
                      o0, o1, o2, o3, o4,
                      idx_v, rows_a, rows_b, sem, wsem_a, wsem_b):
    wid = lax.axis_index("s") * NC + lax.axis_index("c")
    base = wid * ROWS_W
    pltpu.sync_copy(idx_hbm.at[wid], idx_v)
    tbls = [t0, t1, t2, t3, t4]
    outs = [o0, o1, o2, o3, o4]
    bufs = [rows_a, rows_b]
    wsems = [wsem_a, wsem_b]
    wcps = [None, None]
    nhalf = ROWS_W // BUFR
    for p in range(NT * nhalf):
        t, half = p // nhalf, p % nhalf
        buf = bufs[p % 2]
        if wcps[p % 2] is not None:
            wcps[p % 2].wait()
        off = half * BUFR

        @pl.loop(0, BUFR // 16)
        def _(g, t=t, buf=buf, off=off):
            v = idx_v[t, pl.ds(off + g * 16, 16)]
            for l in range(16):
                pltpu.async_copy(
                    tbls[t].at[v[l]], buf.at[g * 16 + l], sem
                )

        pltpu.make_async_copy(tbls[t].at[pl.ds(0, BUFR)], buf, sem).wait()
        wcps[p % 2] = pltpu.async_copy(
            buf, outs[t].at[pl.ds(base + off, BUFR)], wsems[p % 2]
        )
    wcps[0].wait()
    wcps[1].wait()


def _sc_gather(idx_stack, tables):
    mesh = plsc.VectorSubcoreMesh(core_axis_name="c", subcore_axis_name="s")
    out_type = tuple(
        jax.ShapeDtypeStruct((B, ED), jnp.float32) for _ in range(NT)
    )
    k = pl.kernel(
        _sc_gather_kernel,
        out_type=out_type,
        mesh=mesh,
        scratch_types=[
            pltpu.VMEM((NT, ROWS_W), jnp.int32),
            pltpu.VMEM((BUFR, ED), jnp.float32),
            pltpu.VMEM((BUFR, ED), jnp.float32),
            pltpu.SemaphoreType.DMA,
            pltpu.SemaphoreType.DMA,
            pltpu.SemaphoreType.DMA,
        ],
    )
    return k(idx_stack, *tables)


def _mlp_body(g0, g1, g2, g3, g4, w1, b1, w2, b2, w3, b3, out):
    x = jnp.concatenate(
        [g0[...], g1[...], g2[...], g3[...], g4[...]], axis=1
    )
    h = jnp.dot(x, w1[...], preferred_element_type=jnp.float32,
                precision=lax.Precision.HIGHEST)
    h = jnp.maximum(h + b1[...], 0.0)
    h2 = jnp.dot(h, w2[...], preferred_element_type=jnp.float32,
                 precision=lax.Precision.HIGHEST)
    h2 = jnp.maximum(h2 + b2[...], 0.0)
    o = jnp.sum(h2 * w3[...], axis=1) + b3[0, 0]
    out[0, 0, :] = o


def _tc_mlp(gs, W1, b1, W2, b2, W3, b3):
    full = lambda shape: pl.BlockSpec(shape, lambda i: (0, 0))
    out = pl.pallas_call(
        _mlp_body,
        grid=(NB,),
        in_specs=[pl.BlockSpec((BBLK, ED), lambda i: (i, 0))] * NT + [
            full((NT * ED, ED)),
            full((1, ED)),
            full((ED, 32)),
            full((1, 32)),
            full((1, 32)),
            full((1, 1)),
        ],
        out_specs=pl.BlockSpec((1, 1, BBLK), lambda i: (i, 0, 0)),
        out_shape=jax.ShapeDtypeStruct((NB, 1, BBLK), jnp.float32),
    )(*gs, W1, b1.reshape(1, ED), W2, b2.reshape(1, 32),
      W3.reshape(1, 32), b3.reshape(1, 1))
    return out.reshape(-1)


def kernel(user, item, genre, country, tags,
           user_table, item_table, genre_table, country_table, tags_table,
           W1, b1, W2, b2, W3, b3):
    idx = jnp.stack([user, item, genre, country, tags]).astype(jnp.int32)
    idx = idx.reshape(NT, NW, ROWS_W).transpose(1, 0, 2)
    gs = _sc_gather(
        idx, (user_table, item_table, genre_table, country_table, tags_table)
    )
    return _tc_mlp(gs, W1, b1, W2, b2, W3, b3)
